# baseline (speedup 1.0000x reference)
import functools

import jax
import jax.numpy as jnp
from jax import lax
from jax.experimental import pallas as pl
from jax.experimental.pallas import tpu as pltpu
from jax.experimental.pallas import tpu_sc as plsc

L = 2
M = 100000
H = 128
B = 16384

NC = 2
NS = 16
NW = NC * NS

COPY_BLOCK = 2000
N_BLOCKS = (L * M) // COPY_BLOCK

RB_PER_W = (L * B) // NW
CHUNK = 128
N_CH = RB_PER_W // CHUNK


def _tc_copy_body(hsrc, csrc, hdst, cdst):
  hdst[...] = hsrc[...]
  cdst[...] = csrc[...]


_tc_copy = pl.pallas_call(
    _tc_copy_body,
    grid=(N_BLOCKS,),
    in_specs=[
        pl.BlockSpec((COPY_BLOCK, H), lambda i: (i, 0)),
        pl.BlockSpec((COPY_BLOCK, H), lambda i: (i, 0)),
    ],
    out_specs=[
        pl.BlockSpec((COPY_BLOCK, H), lambda i: (i, 0)),
        pl.BlockSpec((COPY_BLOCK, H), lambda i: (i, 0)),
    ],
    out_shape=[
        jax.ShapeDtypeStruct((L * M, H), jnp.float32),
        jax.ShapeDtypeStruct((L * M, H), jnp.float32),
    ],
)


def _sc_gather_body(memh, memc, idx2, hin, cin, gbuf, idxb, sem):
  c = lax.axis_index("c")
  s = lax.axis_index("s")
  base0 = (c * NS + s) * RB_PER_W

  def gather_step(j, carry):
    base = base0 + j * CHUNK
    pltpu.sync_copy(idx2.at[pl.ds(base, CHUNK)], idxb)
    pltpu.async_copy(memh.at[idxb], gbuf, sem).wait()
    pltpu.sync_copy(gbuf, hin.at[pl.ds(base, CHUNK)])
    pltpu.async_copy(memc.at[idxb], gbuf, sem).wait()
    pltpu.sync_copy(gbuf, cin.at[pl.ds(base, CHUNK)])
    return carry

  lax.fori_loop(0, N_CH, gather_step, 0)


_sc_gather = functools.partial(
    pl.kernel,
    out_type=(
        jax.ShapeDtypeStruct((L * B, H), jnp.float32),
        jax.ShapeDtypeStruct((L * B, H), jnp.float32),
    ),
    mesh=plsc.VectorSubcoreMesh(core_axis_name="c", subcore_axis_name="s"),
    scratch_types=[
        pltpu.VMEM((CHUNK, H), jnp.float32),
        pltpu.VMEM((CHUNK,), jnp.int32),
        pltpu.SemaphoreType.DMA,
    ],
)(_sc_gather_body)


def _sc_scatter_body(hv, cv, idx2, src2, outh, outc, sbuf, idxb, srcb, sem):
  c = lax.axis_index("c")
  s = lax.axis_index("s")
  base0 = (c * NS + s) * RB_PER_W

  def scatter_step(j, carry):
    base = base0 + j * CHUNK
    pltpu.sync_copy(idx2.at[pl.ds(base, CHUNK)], idxb)
    pltpu.sync_copy(src2.at[pl.ds(base, CHUNK)], srcb)
    pltpu.async_copy(hv.at[srcb], sbuf, sem).wait()
    pltpu.async_copy(sbuf, outh.at[idxb], sem).wait()
    pltpu.async_copy(cv.at[srcb], sbuf, sem).wait()
    pltpu.async_copy(sbuf, outc.at[idxb], sem).wait()
    return carry

  lax.fori_loop(0, N_CH, scatter_step, 0)


_sc_scatter = functools.partial(
    pl.kernel,
    out_type=(),
    mesh=plsc.VectorSubcoreMesh(core_axis_name="c", subcore_axis_name="s"),
    scratch_types=[
        pltpu.VMEM((CHUNK, H), jnp.float32),
        pltpu.VMEM((CHUNK,), jnp.int32),
        pltpu.VMEM((CHUNK,), jnp.int32),
        pltpu.SemaphoreType.DMA,
    ],
)(_sc_scatter_body)


def kernel(mem_h, mem_c, slots, h_out, c_out):
  slots = slots.astype(jnp.int32)
  iota = lax.iota(jnp.int32, B)
  winner = jnp.full((M,), -1, jnp.int32).at[slots].max(iota)
  src = winner[slots]
  idx2 = jnp.concatenate([slots, slots + M])
  src2 = jnp.concatenate([src, src + B])

  hin, cin = _sc_gather(
      mem_h.reshape(L * M, H), mem_c.reshape(L * M, H), idx2)
  outh0, outc0 = _tc_copy(mem_h.reshape(L * M, H), mem_c.reshape(L * M, H))
  rh = jax.new_ref(outh0)
  rc = jax.new_ref(outc0)
  _sc_scatter(
      h_out.reshape(L * B, H),
      c_out.reshape(L * B, H),
      idx2,
      src2,
      rh,
      rc,
  )
  return (hin.reshape(L, B, H), cin.reshape(L, B, H),
          rh[...].reshape(L, M, H), rc[...].reshape(L, M, H))
